# baseline (speedup 1.0000x reference)
import functools

import jax
import jax.numpy as jnp
from jax import lax
from jax.experimental import pallas as pl
from jax.experimental.pallas import tpu as pltpu
from jax.experimental.pallas import tpu_sc as plsc

N_NODES = 10000
N_EDGES = 320000
D_IN = 128
D_OUT = 128

NC = 2
NS = 16
NW = NC * NS
CHUNK = 64
NCHUNKS = 157
E_PAD = NW * NCHUNKS * CHUNK
PK = 2 * CHUNK
N_PAD = 10240
ROWS_PER_TILE = N_PAD // NS
LANES = 16
NGROUPS = CHUNK // LANES


@functools.partial(
    pl.kernel,
    out_type=jax.ShapeDtypeStruct((NC, N_PAD, D_IN), jnp.float32),
    mesh=plsc.VectorSubcoreMesh(core_axis_name="c", subcore_axis_name="s"),
    scratch_types=[
        pltpu.VMEM((PK,), jnp.int32),
        pltpu.VMEM((PK,), jnp.int32),
        pltpu.VMEM((CHUNK,), jnp.float32),
        pltpu.VMEM((CHUNK,), jnp.float32),
        pltpu.VMEM((CHUNK,), jnp.int32),
        pltpu.VMEM((CHUNK,), jnp.int32),
        pltpu.VMEM((CHUNK, D_IN), jnp.float32),
        pltpu.VMEM((CHUNK, D_IN), jnp.float32),
        pltpu.VMEM_SHARED((N_PAD, D_IN), jnp.float32),
        pltpu.SemaphoreType.DMA,
        pltpu.SemaphoreType.DMA,
        pltpu.SemaphoreType.DMA,
        pltpu.SemaphoreType.DMA,
    ],
)
def _sc_spmm(h_hbm, packed_hbm, val_hbm, out_hbm,
             p0, p1, v0, v1, r0, r1, b0, b1, acc, g0, g1, s0, s1):
    cid = lax.axis_index("c")
    sid = lax.axis_index("s")
    wid = sid * NC + cid
    base0 = wid * NCHUNKS

    zv = jnp.zeros((LANES,), jnp.float32)
    for e in range(CHUNK):
        for j in range(D_IN // LANES):
            b0[e, pl.ds(j * LANES, LANES)] = zv

    def zero_body(i, carry):
        pltpu.sync_copy(
            b0, acc.at[pl.ds(sid * ROWS_PER_TILE + i * CHUNK, CHUNK)])
        return carry

    lax.fori_loop(0, ROWS_PER_TILE // CHUNK, zero_body, 0)
    plsc.subcore_barrier()

    def load_packed(c, pv, vv_ref):
        pltpu.sync_copy(packed_hbm.at[pl.ds((base0 + c) * PK, PK)], pv)
        pltpu.sync_copy(
            val_hbm.at[pl.ds(base0 * CHUNK + c * CHUNK, CHUNK)], vv_ref)

    def copy_row_idx(pv, rv):
        for i in range(NGROUPS):
            rv[pl.ds(i * LANES, LANES)] = pv[pl.ds(CHUNK + i * LANES, LANES)]

    def start_gather(pv, bv, sem):
        pltpu.async_copy(h_hbm.at[pv.at[pl.ds(0, CHUNK)]], bv, sem)

    def wait_gather(pv, bv, sem):
        pltpu.make_async_copy(h_hbm.at[pv.at[pl.ds(0, CHUNK)]], bv, sem).wait()

    def scale(bv, vv_ref):
        def group_body(g, carry):
            v16 = vv_ref[pl.ds(g * LANES, LANES)]
            for l in range(LANES):
                vv = jnp.full((LANES,), v16[l], jnp.float32)
                e = g * LANES + l
                for j in range(D_IN // LANES):
                    sl = pl.ds(j * LANES, LANES)
                    bv[e, sl] = bv[e, sl] * vv
            return carry

        lax.fori_loop(0, NGROUPS, group_body, 0)

    def start_scatter(bv, rv, sem):
        pltpu.async_copy(bv, acc.at[rv], sem, add=True)

    def wait_scatter(bv, rv, sem):
        pltpu.make_async_copy(bv, acc.at[rv], sem).wait()

    def step(pv, vv, rv, bv, gsem, ssem, qv, qvv, qr, qb, qg, qs, c_next):
        wait_scatter(qb, qr, qs)
        load_packed(c_next, qv, qvv)
        copy_row_idx(qv, qr)
        start_gather(qv, qb, qg)
        wait_gather(pv, bv, gsem)
        scale(bv, vv)
        start_scatter(bv, rv, ssem)

    load_packed(0, p0, v0)
    copy_row_idx(p0, r0)
    start_gather(p0, b0, g0)
    load_packed(1, p1, v1)
    copy_row_idx(p1, r1)
    start_gather(p1, b1, g1)
    wait_gather(p0, b0, g0)
    scale(b0, v0)
    start_scatter(b0, r0, s0)

    def pair_body(k, carry):
        step(p1, v1, r1, b1, g1, s1, p0, v0, r0, b0, g0, s0, 2 * k + 2)
        step(p0, v0, r0, b0, g0, s0, p1, v1, r1, b1, g1, s1, 2 * k + 3)
        return carry

    lax.fori_loop(0, (NCHUNKS - 3) // 2, pair_body, 0)

    step(p1, v1, r1, b1, g1, s1, p0, v0, r0, b0, g0, s0, NCHUNKS - 1)
    wait_scatter(b1, r1, s1)
    wait_gather(p0, b0, g0)
    scale(b0, v0)
    start_scatter(b0, r0, s0)
    wait_scatter(b0, r0, s0)

    plsc.subcore_barrier()
    pltpu.sync_copy(acc.at[pl.ds(sid * ROWS_PER_TILE, ROWS_PER_TILE)],
                    out_hbm.at[cid, pl.ds(sid * ROWS_PER_TILE, ROWS_PER_TILE)])


_BM = 1000


def _tc_body(p_ref, w_ref, b_ref, o_ref):
    s = p_ref[0] + p_ref[1]
    acc = jnp.dot(s, w_ref[...], preferred_element_type=jnp.float32)
    o_ref[...] = jnp.maximum(acc + b_ref[...], 0.0)


def _tc_combine(partials, W, bias2d):
    return pl.pallas_call(
        _tc_body,
        grid=(N_NODES // _BM,),
        in_specs=[
            pl.BlockSpec((NC, _BM, D_IN), lambda i: (0, i, 0)),
            pl.BlockSpec((D_IN, D_OUT), lambda i: (0, 0)),
            pl.BlockSpec((1, D_OUT), lambda i: (0, 0)),
        ],
        out_specs=pl.BlockSpec((_BM, D_OUT), lambda i: (i, 0)),
        out_shape=jax.ShapeDtypeStruct((N_NODES, D_OUT), jnp.float32),
    )(partials, W, bias2d)


def kernel(A_edge_index, A_values, H, W, bias):
    row = A_edge_index[0]
    col = A_edge_index[1]
    pad = E_PAD - N_EDGES
    zi = jnp.zeros((pad,), jnp.int32)
    col_p = jnp.concatenate([col, zi])
    row_p = jnp.concatenate([row, zi])
    val_p = jnp.concatenate([A_values, jnp.zeros((pad,), jnp.float32)])
    nchunks_total = E_PAD // CHUNK
    packed = jnp.concatenate(
        [col_p.reshape(nchunks_total, CHUNK),
         row_p.reshape(nchunks_total, CHUNK)], axis=1).reshape(-1)
    partials = _sc_spmm(H, packed, val_p)
    return _tc_combine(partials, W, bias.reshape(1, D_OUT))


# speedup vs baseline: 1.3135x; 1.3135x over previous
import functools

import jax
import jax.numpy as jnp
from jax import lax
from jax.experimental import pallas as pl
from jax.experimental.pallas import tpu as pltpu
from jax.experimental.pallas import tpu_sc as plsc

N_NODES = 10000
N_EDGES = 320000
D_IN = 128
D_OUT = 128

NC = 2
NS = 16
NW = NC * NS
CHUNK = 80
NCHUNKS = 125
E_PAD = NW * NCHUNKS * CHUNK
PK = 2 * CHUNK
N_PAD = 10240
ROWS_PER_TILE = N_PAD // NS
LANES = 16
NGROUPS = CHUNK // LANES


@functools.partial(
    pl.kernel,
    out_type=jax.ShapeDtypeStruct((NC, N_PAD, D_IN), jnp.float32),
    mesh=plsc.VectorSubcoreMesh(core_axis_name="c", subcore_axis_name="s"),
    scratch_types=[
        pltpu.VMEM((PK,), jnp.int32),
        pltpu.VMEM((PK,), jnp.int32),
        pltpu.VMEM((CHUNK,), jnp.float32),
        pltpu.VMEM((CHUNK,), jnp.float32),
        pltpu.VMEM((CHUNK,), jnp.int32),
        pltpu.VMEM((CHUNK,), jnp.int32),
        pltpu.VMEM((CHUNK, D_IN), jnp.float32),
        pltpu.VMEM((CHUNK, D_IN), jnp.float32),
        pltpu.VMEM_SHARED((N_PAD, D_IN), jnp.float32),
        pltpu.SemaphoreType.DMA,
        pltpu.SemaphoreType.DMA,
        pltpu.SemaphoreType.DMA,
        pltpu.SemaphoreType.DMA,
        pltpu.SemaphoreType.DMA,
        pltpu.SemaphoreType.DMA,
    ],
)
def _sc_spmm(h_hbm, packed_hbm, val_hbm, out_hbm,
             p0, p1, v0, v1, r0, r1, b0, b1, acc, g0, g1, s0, s1, i0, i1):
    cid = lax.axis_index("c")
    sid = lax.axis_index("s")
    wid = sid * NC + cid
    base0 = wid * NCHUNKS

    zv = jnp.zeros((LANES,), jnp.float32)
    for e in range(CHUNK):
        for j in range(D_IN // LANES):
            b0[e, pl.ds(j * LANES, LANES)] = zv

    def zero_body(i, carry):
        pltpu.sync_copy(
            b0, acc.at[pl.ds(sid * ROWS_PER_TILE + i * CHUNK, CHUNK)])
        return carry

    lax.fori_loop(0, ROWS_PER_TILE // CHUNK, zero_body, 0)
    plsc.subcore_barrier()

    def start_load_idx(c, pv, vv_ref, sem):
        pltpu.async_copy(packed_hbm.at[pl.ds((base0 + c) * PK, PK)], pv, sem)
        pltpu.async_copy(
            val_hbm.at[pl.ds(base0 * CHUNK + c * CHUNK, CHUNK)], vv_ref, sem)

    def wait_load_idx(c, pv, vv_ref, sem):
        pltpu.make_async_copy(
            packed_hbm.at[pl.ds((base0 + c) * PK, PK)], pv, sem).wait()
        pltpu.make_async_copy(
            val_hbm.at[pl.ds(base0 * CHUNK + c * CHUNK, CHUNK)], vv_ref,
            sem).wait()

    def copy_row_idx(pv, rv):
        for i in range(NGROUPS):
            rv[pl.ds(i * LANES, LANES)] = pv[pl.ds(CHUNK + i * LANES, LANES)]

    def start_gather(pv, bv, sem):
        pltpu.async_copy(h_hbm.at[pv.at[pl.ds(0, CHUNK)]], bv, sem)

    def wait_gather(pv, bv, sem):
        pltpu.make_async_copy(h_hbm.at[pv.at[pl.ds(0, CHUNK)]], bv, sem).wait()

    def scale(bv, vv_ref):
        def group_body(g, carry):
            v16 = vv_ref[pl.ds(g * LANES, LANES)]
            for l in range(LANES):
                vv = jnp.full((LANES,), v16[l], jnp.float32)
                e = g * LANES + l
                for j in range(D_IN // LANES):
                    sl = pl.ds(j * LANES, LANES)
                    bv[e, sl] = bv[e, sl] * vv
            return carry

        lax.fori_loop(0, NGROUPS, group_body, 0)

    def start_scatter(bv, rv, sem):
        pltpu.async_copy(bv, acc.at[rv], sem, add=True)

    def wait_scatter(bv, rv, sem):
        pltpu.make_async_copy(bv, acc.at[rv], sem).wait()

    def step(c, pv, vv, rv, bv, gsem, ssem, isem,
             qv, qvv, qr, qb, qg, qs, qi):
        start_load_idx(c + 1, qv, qvv, qi)
        wait_gather(pv, bv, gsem)
        scale(bv, vv)
        wait_scatter(qb, qr, qs)
        wait_load_idx(c + 1, qv, qvv, qi)
        copy_row_idx(qv, qr)
        start_gather(qv, qb, qg)
        start_scatter(bv, rv, ssem)

    start_load_idx(0, p0, v0, i0)
    wait_load_idx(0, p0, v0, i0)
    copy_row_idx(p0, r0)
    start_gather(p0, b0, g0)
    start_load_idx(1, p1, v1, i1)
    wait_gather(p0, b0, g0)
    scale(b0, v0)
    wait_load_idx(1, p1, v1, i1)
    copy_row_idx(p1, r1)
    start_gather(p1, b1, g1)
    start_scatter(b0, r0, s0)

    def pair_body(k, carry):
        step(2 * k + 1, p1, v1, r1, b1, g1, s1, i1,
             p0, v0, r0, b0, g0, s0, i0)
        step(2 * k + 2, p0, v0, r0, b0, g0, s0, i0,
             p1, v1, r1, b1, g1, s1, i1)
        return carry

    lax.fori_loop(0, (NCHUNKS - 3) // 2, pair_body, 0)

    step(NCHUNKS - 2, p1, v1, r1, b1, g1, s1, i1,
         p0, v0, r0, b0, g0, s0, i0)
    wait_gather(p0, b0, g0)
    scale(b0, v0)
    wait_scatter(b1, r1, s1)
    start_scatter(b0, r0, s0)
    wait_scatter(b0, r0, s0)

    plsc.subcore_barrier()
    pltpu.sync_copy(acc.at[pl.ds(sid * ROWS_PER_TILE, ROWS_PER_TILE)],
                    out_hbm.at[cid, pl.ds(sid * ROWS_PER_TILE, ROWS_PER_TILE)])


_BM = 1000


def _tc_body(p_ref, w_ref, b_ref, o_ref):
    s = p_ref[0] + p_ref[1]
    acc = jnp.dot(s, w_ref[...], preferred_element_type=jnp.float32)
    o_ref[...] = jnp.maximum(acc + b_ref[...], 0.0)


def _tc_combine(partials, W, bias2d):
    return pl.pallas_call(
        _tc_body,
        grid=(N_NODES // _BM,),
        in_specs=[
            pl.BlockSpec((NC, _BM, D_IN), lambda i: (0, i, 0)),
            pl.BlockSpec((D_IN, D_OUT), lambda i: (0, 0)),
            pl.BlockSpec((1, D_OUT), lambda i: (0, 0)),
        ],
        out_specs=pl.BlockSpec((_BM, D_OUT), lambda i: (i, 0)),
        out_shape=jax.ShapeDtypeStruct((N_NODES, D_OUT), jnp.float32),
    )(partials, W, bias2d)


def kernel(A_edge_index, A_values, H, W, bias):
    row = A_edge_index[0]
    col = A_edge_index[1]
    pad = E_PAD - N_EDGES
    zi = jnp.zeros((pad,), jnp.int32)
    col_p = jnp.concatenate([col, zi])
    row_p = jnp.concatenate([row, zi])
    val_p = jnp.concatenate([A_values, jnp.zeros((pad,), jnp.float32)])
    nchunks_total = E_PAD // CHUNK
    packed = jnp.concatenate(
        [col_p.reshape(nchunks_total, CHUNK),
         row_p.reshape(nchunks_total, CHUNK)], axis=1).reshape(-1)
    partials = _sc_spmm(H, packed, val_p)
    return _tc_combine(partials, W, bias.reshape(1, D_OUT))


# speedup vs baseline: 1.6988x; 1.2933x over previous
import functools

import jax
import jax.numpy as jnp
from jax import lax
from jax.experimental import pallas as pl
from jax.experimental.pallas import tpu as pltpu
from jax.experimental.pallas import tpu_sc as plsc

N_NODES = 10000
N_EDGES = 320000
D_IN = 128
D_OUT = 128

NC = 2
NS = 16
NW = NC * NS
CHUNK = 80
NCHUNKS = 125
E_PAD = NW * NCHUNKS * CHUNK
PK = 2 * CHUNK
N_PAD = 10240
ROWS_PER_TILE = N_PAD // NS
LANES = 16
NGROUPS = CHUNK // LANES


@functools.partial(
    pl.kernel,
    out_type=jax.ShapeDtypeStruct((NC, N_PAD, D_IN), jnp.float32),
    mesh=plsc.VectorSubcoreMesh(core_axis_name="c", subcore_axis_name="s"),
    scratch_types=[
        pltpu.VMEM((PK,), jnp.int32),
        pltpu.VMEM((PK,), jnp.int32),
        pltpu.VMEM((CHUNK,), jnp.float32),
        pltpu.VMEM((CHUNK,), jnp.float32),
        pltpu.VMEM((CHUNK,), jnp.int32),
        pltpu.VMEM((CHUNK,), jnp.int32),
        pltpu.VMEM((CHUNK, D_IN), jnp.float32),
        pltpu.VMEM((CHUNK, D_IN), jnp.float32),
        pltpu.VMEM_SHARED((N_PAD, D_IN), jnp.float32),
        pltpu.SemaphoreType.DMA,
        pltpu.SemaphoreType.DMA,
        pltpu.SemaphoreType.DMA,
        pltpu.SemaphoreType.DMA,
        pltpu.SemaphoreType.DMA,
        pltpu.SemaphoreType.DMA,
    ],
)
def _sc_spmm(h_hbm, packed_hbm, val_hbm, out_hbm,
             p0, p1, v0, v1, r0, r1, b0, b1, acc, g0, g1, s0, s1, i0, i1):
    cid = lax.axis_index("c")
    sid = lax.axis_index("s")
    wid = sid * NC + cid
    base0 = wid * NCHUNKS

    zv = jnp.zeros((LANES,), jnp.float32)
    for e in range(CHUNK):
        for j in range(D_IN // LANES):
            b0[e, pl.ds(j * LANES, LANES)] = zv

    def zero_body(i, carry):
        pltpu.sync_copy(
            b0, acc.at[pl.ds(sid * ROWS_PER_TILE + i * CHUNK, CHUNK)])
        return carry

    lax.fori_loop(0, ROWS_PER_TILE // CHUNK, zero_body, 0)
    plsc.subcore_barrier()

    def start_load_idx(c, pv, vv_ref, sem):
        pltpu.async_copy(packed_hbm.at[pl.ds((base0 + c) * PK, PK)], pv, sem)
        pltpu.async_copy(
            val_hbm.at[pl.ds(base0 * CHUNK + c * CHUNK, CHUNK)], vv_ref, sem)

    def wait_load_idx(c, pv, vv_ref, sem):
        pltpu.make_async_copy(
            packed_hbm.at[pl.ds((base0 + c) * PK, PK)], pv, sem).wait()
        pltpu.make_async_copy(
            val_hbm.at[pl.ds(base0 * CHUNK + c * CHUNK, CHUNK)], vv_ref,
            sem).wait()

    def copy_row_idx(pv, rv):
        for i in range(NGROUPS):
            rv[pl.ds(i * LANES, LANES)] = pv[pl.ds(CHUNK + i * LANES, LANES)]

    def start_gather(pv, bv, sem):
        pltpu.async_copy(h_hbm.at[pv.at[pl.ds(0, CHUNK)]], bv, sem)

    def wait_gather(pv, bv, sem):
        pltpu.make_async_copy(h_hbm.at[pv.at[pl.ds(0, CHUNK)]], bv, sem).wait()

    def scale(bv, vv_ref):
        def group_body(g, carry):
            v16 = vv_ref[pl.ds(g * LANES, LANES)]
            for l in range(LANES):
                vv = jnp.full((LANES,), v16[l], jnp.float32)
                e = g * LANES + l
                for j in range(D_IN // LANES):
                    sl = pl.ds(j * LANES, LANES)
                    bv[e, sl] = bv[e, sl] * vv
            return carry

        lax.fori_loop(0, NGROUPS, group_body, 0)

    def start_scatter(bv, rv, sem):
        pltpu.async_copy(bv, acc.at[rv], sem, add=True)

    def wait_scatter(bv, rv, sem):
        pltpu.make_async_copy(bv, acc.at[rv], sem).wait()

    def step(c, pv, vv, rv, bv, gsem, ssem, isem,
             qv, qvv, qr, qb, qg, qs, qi, prefetch=True):
        wait_scatter(qb, qr, qs)
        wait_load_idx(c + 1, qv, qvv, qi)
        copy_row_idx(qv, qr)
        start_gather(qv, qb, qg)
        wait_gather(pv, bv, gsem)
        scale(bv, vv)
        if prefetch:
            start_load_idx(c + 2, pv, vv, isem)
        start_scatter(bv, rv, ssem)

    start_load_idx(0, p0, v0, i0)
    wait_load_idx(0, p0, v0, i0)
    copy_row_idx(p0, r0)
    start_gather(p0, b0, g0)
    start_load_idx(1, p1, v1, i1)
    wait_load_idx(1, p1, v1, i1)
    copy_row_idx(p1, r1)
    start_gather(p1, b1, g1)
    wait_gather(p0, b0, g0)
    scale(b0, v0)
    start_load_idx(2, p0, v0, i0)
    start_scatter(b0, r0, s0)

    def pair_body(k, carry):
        step(2 * k + 1, p1, v1, r1, b1, g1, s1, i1,
             p0, v0, r0, b0, g0, s0, i0)
        step(2 * k + 2, p0, v0, r0, b0, g0, s0, i0,
             p1, v1, r1, b1, g1, s1, i1)
        return carry

    lax.fori_loop(0, (NCHUNKS - 3) // 2, pair_body, 0)

    step(NCHUNKS - 2, p1, v1, r1, b1, g1, s1, i1,
         p0, v0, r0, b0, g0, s0, i0, prefetch=False)
    wait_gather(p0, b0, g0)
    scale(b0, v0)
    wait_scatter(b1, r1, s1)
    start_scatter(b0, r0, s0)
    wait_scatter(b0, r0, s0)

    plsc.subcore_barrier()
    pltpu.sync_copy(acc.at[pl.ds(sid * ROWS_PER_TILE, ROWS_PER_TILE)],
                    out_hbm.at[cid, pl.ds(sid * ROWS_PER_TILE, ROWS_PER_TILE)])


_BM = 1000


def _tc_body(p_ref, w_ref, b_ref, o_ref):
    s = p_ref[0] + p_ref[1]
    acc = jnp.dot(s, w_ref[...], preferred_element_type=jnp.float32)
    o_ref[...] = jnp.maximum(acc + b_ref[...], 0.0)


def _tc_combine(partials, W, bias2d):
    return pl.pallas_call(
        _tc_body,
        grid=(N_NODES // _BM,),
        in_specs=[
            pl.BlockSpec((NC, _BM, D_IN), lambda i: (0, i, 0)),
            pl.BlockSpec((D_IN, D_OUT), lambda i: (0, 0)),
            pl.BlockSpec((1, D_OUT), lambda i: (0, 0)),
        ],
        out_specs=pl.BlockSpec((_BM, D_OUT), lambda i: (i, 0)),
        out_shape=jax.ShapeDtypeStruct((N_NODES, D_OUT), jnp.float32),
    )(partials, W, bias2d)


def kernel(A_edge_index, A_values, H, W, bias):
    row = A_edge_index[0]
    col = A_edge_index[1]
    pad = E_PAD - N_EDGES
    zi = jnp.zeros((pad,), jnp.int32)
    col_p = jnp.concatenate([col, zi])
    row_p = jnp.concatenate([row, zi])
    val_p = jnp.concatenate([A_values, jnp.zeros((pad,), jnp.float32)])
    nchunks_total = E_PAD // CHUNK
    packed = jnp.concatenate(
        [col_p.reshape(nchunks_total, CHUNK),
         row_p.reshape(nchunks_total, CHUNK)], axis=1).reshape(-1)
    partials = _sc_spmm(H, packed, val_p)
    return _tc_combine(partials, W, bias.reshape(1, D_OUT))
